# baseline (speedup 1.0000x reference)
import functools

import jax
import jax.numpy as jnp
from jax import lax
from jax.experimental import pallas as pl
from jax.experimental.pallas import tpu as pltpu
from jax.experimental.pallas import tpu_sc as plsc

VOCAB = 1000
EMBED_DIM = 128
BATCH = 4096
HIST_LEN = 50
HIST_PAD = 56

_INFO = plsc.get_sparse_core_info()
NC, NS = _INFO.num_cores, _INFO.num_subcores
NW = NC * NS
KSPLIT = 4
B_CALL = BATCH // KSPLIT
B_PER_W = B_CALL // NW
NBUF = 8
NGROUP = B_PER_W // NBUF
BB = 32


def _build_sc_kernel():
    mesh = plsc.VectorSubcoreMesh(core_axis_name="c", subcore_axis_name="s")

    @functools.partial(
        pl.kernel,
        mesh=mesh,
        out_type=jax.ShapeDtypeStruct((B_CALL, HIST_PAD, EMBED_DIM),
                                      jnp.float32),
        scratch_types=[
            pltpu.VMEM((B_PER_W, HIST_PAD), jnp.int32),
            pltpu.VMEM((NBUF, HIST_PAD, EMBED_DIM), jnp.float32),
            pltpu.VMEM_SHARED((VOCAB, EMBED_DIM), jnp.float32),
        ]
        + [pltpu.SemaphoreType.DMA] * (2 * NBUF),
    )
    def gather_kernel(x_hbm, table_hbm, out_hbm, idx_v, rows_v, table_sp,
                      *sems):
        gsems, osems = sems[:NBUF], sems[NBUF:]
        sid = lax.axis_index("s")
        wid = sid * NC + lax.axis_index("c")
        b0 = wid * B_PER_W

        @pl.when(sid == 0)
        def _stage_table():
            pltpu.sync_copy(table_hbm, table_sp)

        pltpu.sync_copy(x_hbm.at[wid], idx_v)
        plsc.subcore_barrier()

        def gather(i, b):
            return pltpu.make_async_copy(
                table_sp.at[idx_v.at[i, pl.ds(0, HIST_LEN)]],
                rows_v.at[b, pl.ds(0, HIST_LEN)], gsems[b])

        def out_copy(i, b):
            return pltpu.make_async_copy(
                rows_v.at[b], out_hbm.at[b0 + i], osems[b])

        for b in range(NBUF):
            gather(b, b).start()

        def body(g, carry):
            i0 = g * NBUF
            for b in range(NBUF):
                gather(i0 + b, b).wait()
                out_copy(i0 + b, b).start()
            for b in range(NBUF):
                out_copy(i0 + b, b).wait()
                gather(i0 + NBUF + b, b).start()
            return carry

        lax.fori_loop(0, NGROUP - 1, body, 0)

        il = (NGROUP - 1) * NBUF
        for b in range(NBUF):
            gather(il + b, b).wait()
            out_copy(il + b, b).start()
        for b in range(NBUF):
            out_copy(il + b, b).wait()

    return gather_kernel


_SC_KERNEL = _build_sc_kernel()


def _unpad_body(part_ref, prev_ref, out_ref):
    out_ref[...] = part_ref[:, :HIST_LEN, :]


def _unpad_into(part, prev, quarter):
    return pl.pallas_call(
        _unpad_body,
        grid=(B_CALL // BB,),
        in_specs=[
            pl.BlockSpec((BB, HIST_PAD, EMBED_DIM), lambda i: (i, 0, 0)),
            pl.BlockSpec(memory_space=pl.ANY),
        ],
        out_specs=pl.BlockSpec(
            (BB, HIST_LEN, EMBED_DIM),
            lambda i, q=quarter: (i + q * (B_CALL // BB), 0, 0)),
        out_shape=jax.ShapeDtypeStruct((BATCH, HIST_LEN, EMBED_DIM),
                                       jnp.float32),
        input_output_aliases={1: 0},
    )(part, prev)


def _unpad_first(part):
    return pl.pallas_call(
        _unpad_body,
        grid=(B_CALL // BB,),
        in_specs=[
            pl.BlockSpec((BB, HIST_PAD, EMBED_DIM), lambda i: (i, 0, 0)),
            pl.BlockSpec(memory_space=pl.ANY),
        ],
        out_specs=pl.BlockSpec(
            (BB, HIST_LEN, EMBED_DIM), lambda i: (i, 0, 0)),
        out_shape=jax.ShapeDtypeStruct((BATCH, HIST_LEN, EMBED_DIM),
                                       jnp.float32),
    )(part, part)


def kernel(x, table):
    idx = x.astype(jnp.int32)
    idx = jnp.pad(idx, ((0, 0), (0, HIST_PAD - HIST_LEN)))
    idx = idx.reshape(KSPLIT, NW, B_PER_W, HIST_PAD)
    parts = [_SC_KERNEL(idx[k], table) for k in range(KSPLIT)]
    out = _unpad_first(parts[0])
    for k in range(1, KSPLIT):
        out = _unpad_into(parts[k], out, k)
    return out


# speedup vs baseline: 1.2397x; 1.2397x over previous
import functools

import jax
import jax.numpy as jnp
from jax import lax
from jax.experimental import pallas as pl
from jax.experimental.pallas import tpu as pltpu
from jax.experimental.pallas import tpu_sc as plsc

VOCAB = 1000
EMBED_DIM = 128
BATCH = 4096
HIST_LEN = 50
HIST_PAD = 56

_INFO = plsc.get_sparse_core_info()
NC, NS = _INFO.num_cores, _INFO.num_subcores
NW = NC * NS
B_PER_W = BATCH // NW
BCHUNK = 8
NCHUNK = B_PER_W // BCHUNK
NBUF = 2
NGROUP = NCHUNK // NBUF


def _build_kernel():
    mesh = plsc.VectorSubcoreMesh(core_axis_name="c", subcore_axis_name="s")

    @functools.partial(
        pl.kernel,
        mesh=mesh,
        out_type=jax.ShapeDtypeStruct((BATCH, HIST_LEN, EMBED_DIM),
                                      jnp.float32),
        scratch_types=[
            pltpu.VMEM((B_PER_W, HIST_PAD), jnp.int32),
            pltpu.VMEM((NBUF, BCHUNK, HIST_LEN, EMBED_DIM), jnp.float32),
        ]
        + [pltpu.SemaphoreType.DMA] * (2 * NBUF),
    )
    def gather_kernel(x_hbm, table_hbm, out_hbm, idx_v, rows_v,
                      *sems):
        gsems, osems = sems[:NBUF], sems[NBUF:]
        sid = lax.axis_index("s")
        wid = sid * NC + lax.axis_index("c")
        b0 = wid * B_PER_W

        pltpu.sync_copy(x_hbm.at[wid], idx_v)

        def gathers(i, b):
            for j in range(BCHUNK):
                pltpu.make_async_copy(
                    table_hbm.at[idx_v.at[i * BCHUNK + j, pl.ds(0, HIST_LEN)]],
                    rows_v.at[b, j], gsems[b]).start()

        def wait_gathers(i, b):
            for j in range(BCHUNK):
                pltpu.make_async_copy(
                    table_hbm.at[idx_v.at[i * BCHUNK + j, pl.ds(0, HIST_LEN)]],
                    rows_v.at[b, j], gsems[b]).wait()

        def out_copy(i, b):
            return pltpu.make_async_copy(
                rows_v.at[b], out_hbm.at[pl.ds(b0 + i * BCHUNK, BCHUNK)],
                osems[b])

        for b in range(NBUF):
            gathers(b, b)

        def body(g, carry):
            i0 = g * NBUF
            for b in range(NBUF):
                wait_gathers(i0 + b, b)
                out_copy(i0 + b, b).start()
            for b in range(NBUF):
                out_copy(i0 + b, b).wait()
                gathers(i0 + NBUF + b, b)
            return carry

        lax.fori_loop(0, NGROUP - 1, body, 0)

        il = (NGROUP - 1) * NBUF
        for b in range(NBUF):
            wait_gathers(il + b, b)
            out_copy(il + b, b).start()
        for b in range(NBUF):
            out_copy(il + b, b).wait()

    return gather_kernel


_KERNEL = _build_kernel()


def kernel(x, table):
    idx = x.astype(jnp.int32)
    idx = jnp.pad(idx, ((0, 0), (0, HIST_PAD - HIST_LEN)))
    idx = idx.reshape(NW, B_PER_W, HIST_PAD)
    return _KERNEL(idx, table)


# speedup vs baseline: 1.8691x; 1.5078x over previous
import functools

import jax
import jax.numpy as jnp
from jax import lax
from jax.experimental import pallas as pl
from jax.experimental.pallas import tpu as pltpu
from jax.experimental.pallas import tpu_sc as plsc

VOCAB = 1000
EMBED_DIM = 128
BATCH = 4096
HIST_LEN = 50
HIST_PAD = 56

_INFO = plsc.get_sparse_core_info()
NC, NS = _INFO.num_cores, _INFO.num_subcores
NW = NC * NS
B_PER_W = BATCH // NW
NBUF = 8
NGROUP = B_PER_W // NBUF


def _build_kernel():
    mesh = plsc.VectorSubcoreMesh(core_axis_name="c", subcore_axis_name="s")

    @functools.partial(
        pl.kernel,
        mesh=mesh,
        out_type=jax.ShapeDtypeStruct((BATCH, HIST_LEN, EMBED_DIM),
                                      jnp.float32),
        scratch_types=[
            pltpu.VMEM((B_PER_W, HIST_PAD), jnp.int32),
            pltpu.VMEM((NBUF, HIST_LEN, EMBED_DIM), jnp.float32),
            pltpu.VMEM_SHARED((VOCAB, EMBED_DIM), jnp.float32),
        ]
        + [pltpu.SemaphoreType.DMA] * (2 * NBUF),
    )
    def gather_kernel(x_hbm, table_hbm, out_hbm, idx_v, rows_v, table_sp,
                      *sems):
        gsems, osems = sems[:NBUF], sems[NBUF:]
        sid = lax.axis_index("s")
        wid = sid * NC + lax.axis_index("c")
        b0 = wid * B_PER_W

        @pl.when(sid == 0)
        def _stage_table():
            pltpu.sync_copy(table_hbm, table_sp)

        pltpu.sync_copy(x_hbm.at[wid], idx_v)
        plsc.subcore_barrier()

        def gather(i, b):
            return pltpu.make_async_copy(
                table_sp.at[idx_v.at[i, pl.ds(0, HIST_LEN)]],
                rows_v.at[b], gsems[b])

        def out_copy(i, b):
            return pltpu.make_async_copy(
                rows_v.at[b], out_hbm.at[b0 + i], osems[b])

        for b in range(NBUF):
            gather(b, b).start()

        def body(g, carry):
            i0 = g * NBUF
            for b in range(NBUF):
                gather(i0 + b, b).wait()
                out_copy(i0 + b, b).start()
            for b in range(NBUF):
                out_copy(i0 + b, b).wait()
                gather(i0 + NBUF + b, b).start()
            return carry

        lax.fori_loop(0, NGROUP - 1, body, 0)

        il = (NGROUP - 1) * NBUF
        for b in range(NBUF):
            gather(il + b, b).wait()
            out_copy(il + b, b).start()
        for b in range(NBUF):
            out_copy(il + b, b).wait()

    return gather_kernel


_KERNEL = _build_kernel()


def kernel(x, table):
    idx = x.astype(jnp.int32)
    idx = jnp.pad(idx, ((0, 0), (0, HIST_PAD - HIST_LEN)))
    idx = idx.reshape(NW, B_PER_W, HIST_PAD)
    return _KERNEL(idx, table)
